# baseline (speedup 1.0000x reference)
import functools

import jax
import jax.numpy as jnp
from jax import lax
from jax.experimental import pallas as pl
from jax.experimental.pallas import tpu as pltpu
from jax.experimental.pallas import tpu_sc as plsc

_BATCH = 4096
_FIELDS = 26
_EMBED = 32
_TOTAL = _BATCH * _FIELDS

_NC = 2
_NS = 16
_NW = _NC * _NS
_PER_W = _TOTAL // _NW
_CHUNK = 128
_CHUNKS_PER_W = _PER_W // _CHUNK


def _gather_body(idx_hbm, tbl_hbm, out_hbm, idx_v, rows_v, sem):
    wid = lax.axis_index("s") * _NC + lax.axis_index("c")
    pltpu.sync_copy(idx_hbm.at[pl.ds(wid * _PER_W, _PER_W)], idx_v)
    copies = []
    for j in range(_CHUNKS_PER_W):
        copies.append(
            pltpu.async_copy(
                tbl_hbm.at[idx_v.at[pl.ds(j * _CHUNK, _CHUNK)]],
                rows_v.at[pl.ds(j * _CHUNK, _CHUNK)],
                sem,
            )
        )
    for c in copies:
        c.wait()
    pltpu.sync_copy(rows_v, out_hbm.at[pl.ds(wid * _PER_W, _PER_W)])


@jax.jit
def _gather(idx2d, weight):
    mesh = plsc.VectorSubcoreMesh(core_axis_name="c", subcore_axis_name="s")
    run = pl.kernel(
        _gather_body,
        out_type=jax.ShapeDtypeStruct((_TOTAL, _EMBED), jnp.float32),
        mesh=mesh,
        compiler_params=pltpu.CompilerParams(use_tc_tiling_on_sc=False),
        scratch_types=[
            pltpu.VMEM((_PER_W,), jnp.int32),
            pltpu.VMEM((_PER_W, _EMBED), jnp.float32),
            pltpu.SemaphoreType.DMA,
        ],
    )
    return run(idx2d, weight)


def kernel(input_tensor, weight):
    idx = input_tensor
    if idx.ndim == 1:
        idx = idx[None, :]
    lead_shape = idx.shape
    out = _gather(idx.reshape(_TOTAL), weight)
    return out.reshape(*lead_shape, _EMBED)


# speedup vs baseline: 1.4797x; 1.4797x over previous
import functools

import jax
import jax.numpy as jnp
from jax import lax
from jax.experimental import pallas as pl
from jax.experimental.pallas import tpu as pltpu
from jax.experimental.pallas import tpu_sc as plsc

_BATCH = 4096
_FIELDS = 26
_EMBED = 32
_N = 1000000
_TOTAL = _BATCH * _FIELDS

_BK = 8192
_NBLK = (_N + _BK - 1) // _BK
_PACKED_ROWS = _NBLK * (_BK // 4)

_NW = 32
_PER_W = _TOTAL // _NW
_CHUNKS = _PER_W // 128


def _t_body(x_ref, o_ref):
    xT = x_ref[...].T
    q = _BK // 4
    for g in range(4):
        o_ref[:, g * 32:(g + 1) * 32] = xT[g * q:(g + 1) * q, :]


def _tc_pack(wT):
    return pl.pallas_call(
        _t_body,
        out_shape=jax.ShapeDtypeStruct((_PACKED_ROWS, 128), jnp.float32),
        grid=(_NBLK,),
        in_specs=[pl.BlockSpec((32, _BK), lambda j: (0, j))],
        out_specs=pl.BlockSpec((_BK // 4, 128), lambda j: (j, 0)),
    )(wT)


def _bcast(vec, k):
    return lax.gather(
        vec, jnp.full((16, 1), k, jnp.int32),
        lax.GatherDimensionNumbers(
            offset_dims=(), collapsed_slice_dims=(0,), start_index_map=(0,)),
        slice_sizes=(1,),
        mode=lax.GatherScatterMode.PROMISE_IN_BOUNDS)


def _g_body(idxf, w4, out, idxv, rivv, rows_v, opbuf, sem):
    w = lax.axis_index("s") * 2 + lax.axis_index("c")
    base = pl.multiple_of(w * _PER_W, 8)
    pltpu.sync_copy(idxf.at[pl.ds(base, _PER_W)], idxv)
    for m in range(_PER_W // 16):
        v = idxv[pl.ds(m * 16, 16)]
        rivv[pl.ds(m * 16, 16)] = ((v >> 13) << 11) + (v & 2047)

    def chunk(j, carry):
        c0 = pl.multiple_of(j * 128, 8)
        pltpu.async_copy(w4.at[rivv.at[pl.ds(c0, 128)]], rows_v, sem).wait()
        for mm in range(8):
            g16 = (idxv[pl.ds(c0 + mm * 16, 16)] >> 11) & 3
            oh = jnp.int32(1) << g16
            mf = [((oh >> g) & 1).astype(jnp.float32) for g in range(4)]
            for k in range(16):
                l = mm * 16 + k
                mb = [_bcast(mf[g], k) for g in range(4)]
                dst = (l % 4) * 32
                for h in (0, 16):
                    acc = mb[0] * rows_v[l, pl.ds(h, 16)]
                    acc = acc + mb[1] * rows_v[l, pl.ds(32 + h, 16)]
                    acc = acc + mb[2] * rows_v[l, pl.ds(64 + h, 16)]
                    acc = acc + mb[3] * rows_v[l, pl.ds(96 + h, 16)]
                    opbuf[l // 4, pl.ds(dst + h, 16)] = acc
        o0 = pl.multiple_of(w * (_PER_W // 4) + j * 32, 8)
        pltpu.sync_copy(opbuf, out.at[pl.ds(o0, 32), :])
        return carry

    lax.fori_loop(0, _CHUNKS, chunk, 0)


@jax.jit
def _run(idxf, wT):
    w4 = _tc_pack(wT)
    mesh = plsc.VectorSubcoreMesh(core_axis_name="c", subcore_axis_name="s")
    gather = pl.kernel(
        _g_body,
        out_type=jax.ShapeDtypeStruct((_TOTAL // 4, 128), jnp.float32),
        mesh=mesh,
        compiler_params=pltpu.CompilerParams(use_tc_tiling_on_sc=True),
        scratch_types=[
            pltpu.VMEM((_PER_W,), jnp.int32),
            pltpu.VMEM((_PER_W,), jnp.int32),
            pltpu.VMEM((128, 128), jnp.float32),
            pltpu.VMEM((32, 128), jnp.float32),
            pltpu.SemaphoreType.DMA,
        ],
    )
    return gather(idxf, w4)


def kernel(input_tensor, weight):
    idx = input_tensor
    if idx.ndim == 1:
        idx = idx[None, :]
    opack = _run(idx.reshape(-1), weight.T)
    return opack.reshape(*idx.shape, _EMBED)


# speedup vs baseline: 1.5020x; 1.0151x over previous
import functools

import jax
import jax.numpy as jnp
from jax import lax
from jax.experimental import pallas as pl
from jax.experimental.pallas import tpu as pltpu
from jax.experimental.pallas import tpu_sc as plsc

_BATCH = 4096
_FIELDS = 26
_EMBED = 32
_N = 1000000
_TOTAL = _BATCH * _FIELDS

_BK = 32768
_NBLK = (_N + _BK - 1) // _BK
_PACKED_ROWS = _NBLK * (_BK // 4)

_Q = _BK // 4
_SH_BLK = _BK.bit_length() - 1
_SH_G = _Q.bit_length() - 1

_NW = 32
_PER_W = _TOTAL // _NW
_CHUNKS = _PER_W // 128


def _t_body(x_ref, o_ref):
    xT = x_ref[...].T
    q = _BK // 4
    for g in range(4):
        o_ref[:, g * 32:(g + 1) * 32] = xT[g * q:(g + 1) * q, :]


def _tc_pack(wT):
    return pl.pallas_call(
        _t_body,
        out_shape=jax.ShapeDtypeStruct((_PACKED_ROWS, 128), jnp.float32),
        grid=(_NBLK,),
        in_specs=[pl.BlockSpec((32, _BK), lambda j: (0, j))],
        out_specs=pl.BlockSpec((_BK // 4, 128), lambda j: (j, 0)),
    )(wT)


def _bcast(vec, k):
    return lax.gather(
        vec, jnp.full((16, 1), k, jnp.int32),
        lax.GatherDimensionNumbers(
            offset_dims=(), collapsed_slice_dims=(0,), start_index_map=(0,)),
        slice_sizes=(1,),
        mode=lax.GatherScatterMode.PROMISE_IN_BOUNDS)


def _g_body(idxf, w4, out, idxv, rivv, rows_v, opbuf, sem):
    w = lax.axis_index("s") * 2 + lax.axis_index("c")
    base = pl.multiple_of(w * _PER_W, 8)
    pltpu.sync_copy(idxf.at[pl.ds(base, _PER_W)], idxv)
    for m in range(_PER_W // 16):
        v = idxv[pl.ds(m * 16, 16)]
        rivv[pl.ds(m * 16, 16)] = ((v >> _SH_BLK) << _SH_G) + (v & (_Q - 1))

    def chunk(j, carry):
        c0 = pl.multiple_of(j * 128, 8)
        pltpu.async_copy(w4.at[rivv.at[pl.ds(c0, 128)]], rows_v, sem).wait()
        for mm in range(8):
            g16 = (idxv[pl.ds(c0 + mm * 16, 16)] >> _SH_G) & 3
            oh = jnp.int32(1) << g16
            mf = [((oh >> g) & 1).astype(jnp.float32) for g in range(4)]
            for k in range(16):
                l = mm * 16 + k
                mb = [_bcast(mf[g], k) for g in range(4)]
                dst = (l % 4) * 32
                for h in (0, 16):
                    acc = mb[0] * rows_v[l, pl.ds(h, 16)]
                    acc = acc + mb[1] * rows_v[l, pl.ds(32 + h, 16)]
                    acc = acc + mb[2] * rows_v[l, pl.ds(64 + h, 16)]
                    acc = acc + mb[3] * rows_v[l, pl.ds(96 + h, 16)]
                    opbuf[l // 4, pl.ds(dst + h, 16)] = acc
        o0 = pl.multiple_of(w * (_PER_W // 4) + j * 32, 8)
        pltpu.sync_copy(opbuf, out.at[pl.ds(o0, 32), :])
        return carry

    lax.fori_loop(0, _CHUNKS, chunk, 0)


@jax.jit
def _run(idxf, wT):
    w4 = _tc_pack(wT)
    mesh = plsc.VectorSubcoreMesh(core_axis_name="c", subcore_axis_name="s")
    gather = pl.kernel(
        _g_body,
        out_type=jax.ShapeDtypeStruct((_TOTAL // 4, 128), jnp.float32),
        mesh=mesh,
        compiler_params=pltpu.CompilerParams(use_tc_tiling_on_sc=True),
        scratch_types=[
            pltpu.VMEM((_PER_W,), jnp.int32),
            pltpu.VMEM((_PER_W,), jnp.int32),
            pltpu.VMEM((128, 128), jnp.float32),
            pltpu.VMEM((32, 128), jnp.float32),
            pltpu.SemaphoreType.DMA,
        ],
    )
    return gather(idxf, w4)


def kernel(input_tensor, weight):
    idx = input_tensor
    if idx.ndim == 1:
        idx = idx[None, :]
    opack = _run(idx.reshape(-1), weight.T)
    return opack.reshape(*idx.shape, _EMBED)
